# baseline (speedup 1.0000x reference)
import functools

import jax
import jax.numpy as jnp
from jax import lax
from jax.experimental import pallas as pl
from jax.experimental.pallas import tpu as pltpu
from jax.experimental.pallas import tpu_sc as plsc

NUM_CORES = 2
NUM_SUBCORES = 16
NUM_WORKERS = NUM_CORES * NUM_SUBCORES
LANE = 128


def _emb_lookup(hist: int, emb_dim: int, n_bblk: int, vocab: int):
    n_dblk = emb_dim // 8
    mesh = plsc.VectorSubcoreMesh(core_axis_name="c", subcore_axis_name="s")

    @functools.partial(
        pl.kernel,
        mesh=mesh,
        out_type=jax.ShapeDtypeStruct(
            (hist, n_dblk, n_bblk, 8, LANE), jnp.float32
        ),
        scratch_types=[
            pltpu.VMEM((LANE,), jnp.int32),
            pltpu.VMEM((LANE,), jnp.int32),
            pltpu.VMEM((emb_dim, LANE), jnp.float32),
            pltpu.VMEM((emb_dim, LANE), jnp.float32),
            pltpu.SemaphoreType.DMA,
            pltpu.SemaphoreType.DMA,
            pltpu.SemaphoreType.DMA,
            pltpu.SemaphoreType.DMA,
        ],
        compiler_params=pltpu.CompilerParams(
            use_tc_tiling_on_sc=False, needs_layout_passes=False
        ),
    )
    def body(idx_hbm, table_hbm, out_hbm,
             idx_a, idx_b, pl_a, pl_b, sga, sgb, soa, sob):
        w = lax.axis_index("s") * NUM_CORES + lax.axis_index("c")
        slots = ((idx_a, pl_a, sga, soa),
                 (idx_b, pl_b, sgb, sob))

        def stage(h, slot):
            idx_v, pl_v, sg, _ = slot
            pltpu.sync_copy(idx_hbm.at[h // 8, w, h % 8], idx_v)
            for d in range(emb_dim):
                pltpu.async_copy(table_hbm.at[d].at[idx_v], pl_v.at[d], sg)

        def gather_wait(slot):
            idx_v, pl_v, sg, _ = slot
            for d in range(emb_dim):
                pltpu.make_async_copy(
                    table_hbm.at[d].at[idx_v], pl_v.at[d], sg
                ).wait()

        def put(h, slot):
            _, pl_v, _, so = slot
            for g in range(n_dblk):
                pltpu.async_copy(
                    pl_v.at[pl.ds(8 * g, 8), :], out_hbm.at[h, g, w], so
                )

        def put_wait(h, slot):
            _, pl_v, _, so = slot
            for g in range(n_dblk):
                pltpu.make_async_copy(
                    pl_v.at[pl.ds(8 * g, 8), :], out_hbm.at[h, g, w], so
                ).wait()

        stage(0, slots[0])

        def outer(o, carry):
            for b in (0, 1):
                h = o * 2 + b
                nxt = h + 1

                gather_wait(slots[b])
                put(h, slots[b])

                @pl.when(nxt < hist)
                def _():
                    @pl.when(h >= 1)
                    def _():
                        put_wait(h - 1, slots[1 - b])

                    stage(nxt, slots[1 - b])
            return carry

        lax.fori_loop(0, hist // 2, outer, 0)
        put_wait(hist - 2, slots[0])
        put_wait(hist - 1, slots[1])

    return body


def kernel(indices, table):
    batch, hist = indices.shape
    vocab, emb_dim = table.shape
    n_bblk = batch // LANE
    hist_pad = -(-hist // 8) * 8
    n_hblk = hist_pad // 8
    idx_p = jnp.pad(indices.astype(jnp.int32), ((0, 0), (0, hist_pad - hist)))
    idx4 = idx_p.T.reshape(n_hblk, 8, n_bblk, LANE).transpose(0, 2, 1, 3)
    table_f = table.T
    out5 = _emb_lookup(hist, emb_dim, n_bblk, vocab)(idx4, table_f)
    out = out5.transpose(2, 4, 0, 1, 3).reshape(batch, hist, emb_dim)
    return out


# speedup vs baseline: 4.8202x; 4.8202x over previous
import functools

import jax
import jax.numpy as jnp
from jax import lax
from jax.experimental import pallas as pl
from jax.experimental.pallas import tpu as pltpu
from jax.experimental.pallas import tpu_sc as plsc

NUM_CORES = 2
NUM_SUBCORES = 16
NUM_WORKERS = NUM_CORES * NUM_SUBCORES
LANE = 128
GRP = 4


def _detile(vocab: int, emb_dim: int):
    n_cols = vocab // LANE
    n_grow = emb_dim // GRP
    mesh = plsc.VectorSubcoreMesh(core_axis_name="c", subcore_axis_name="s")

    @functools.partial(
        pl.kernel,
        mesh=mesh,
        out_type=jax.ShapeDtypeStruct((vocab // GRP, LANE), jnp.float32),
        scratch_types=[
            pltpu.VMEM((emb_dim, LANE), jnp.float32),
            pltpu.VMEM((emb_dim, LANE), jnp.float32),
            pltpu.VMEM((n_grow, LANE), jnp.float32),
            pltpu.VMEM((n_grow, LANE), jnp.float32),
            pltpu.SemaphoreType.DMA,
            pltpu.SemaphoreType.DMA,
            pltpu.SemaphoreType.DMA,
            pltpu.SemaphoreType.DMA,
        ],
        compiler_params=pltpu.CompilerParams(
            use_tc_tiling_on_sc=True, needs_layout_passes=False
        ),
    )
    def body(tab_hbm, tail_hbm, t2_hbm, in_a, in_b, tr_a, tr_b,
             sia, sib, soa, sob):
        w = lax.axis_index("s") * NUM_CORES + lax.axis_index("c")
        slots = ((in_a, tr_a, sia, soa), (in_b, tr_b, sib, sob))
        n_iter = -(-n_cols // NUM_WORKERS)

        def col(k):
            return w + k * NUM_WORKERS

        def fetch(k, slot):
            in_v, _, si, _ = slot
            pltpu.async_copy(tab_hbm.at[:, pl.ds(col(k) * LANE, LANE)], in_v, si)

        def fetch_wait(k, slot):
            in_v, _, si, _ = slot
            pltpu.make_async_copy(
                tab_hbm.at[:, pl.ds(col(k) * LANE, LANE)], in_v, si
            ).wait()

        def put(k, slot):
            _, tr_v, _, so = slot
            pltpu.async_copy(tr_v, t2_hbm.at[pl.ds(col(k) * n_grow, n_grow)], so)

        def put_wait(k, slot):
            _, tr_v, _, so = slot
            pltpu.make_async_copy(
                tr_v, t2_hbm.at[pl.ds(col(k) * n_grow, n_grow)], so
            ).wait()

        def transpose(slot):
            _, tr_v, _, _ = slot
            in_v = slot[0]
            base = lax.iota(jnp.int32, 16)
            rows = [(base + j0 * 16) // GRP for j0 in range(LANE // 16)]
            cols = [((base + j0 * 16) % GRP) * emb_dim for j0 in range(LANE // 16)]

            @plsc.parallel_loop(0, emb_dim, unroll=4)
            def _(d):
                for j0 in range(LANE // 16):
                    v = in_v[d, pl.ds(j0 * 16, 16)]
                    plsc.store_scatter(tr_v, [rows[j0], cols[j0] + d], v)

        @pl.when(col(0) < n_cols)
        def _():
            fetch(0, slots[0])

        def outer(o, carry):
            for b in (0, 1):
                k = o * 2 + b

                @pl.when(col(k) < n_cols)
                def _():
                    @pl.when(col(k + 1) < n_cols)
                    def _():
                        fetch(k + 1, slots[1 - b])

                    fetch_wait(k, slots[b])

                    @pl.when(k >= 2)
                    def _():
                        put_wait(k - 2, slots[b])

                    transpose(slots[b])
                    put(k, slots[b])
            return carry

        n_outer = -(-n_iter // 2)
        lax.fori_loop(0, n_outer, outer, 0)
        for b in (0, 1):
            k_last = n_iter - 2 + b
            last = slots[k_last % 2]

            @pl.when(col(k_last) < n_cols)
            def _():
                put_wait(k_last, last)

        n_tail = (vocab - n_cols * LANE) // GRP
        @pl.when(w == NUM_WORKERS - 1)
        def _():
            pltpu.sync_copy(tail_hbm, t2_hbm.at[pl.ds(n_cols * n_grow, n_tail)])

    return body


def _emb_lookup(hist: int, emb_dim: int, n_bblk: int):
    n_dblk = emb_dim // 8
    mesh = plsc.VectorSubcoreMesh(core_axis_name="c", subcore_axis_name="s")

    @functools.partial(
        pl.kernel,
        mesh=mesh,
        out_type=jax.ShapeDtypeStruct(
            (hist, n_dblk, n_bblk, 8, LANE), jnp.float32
        ),
        scratch_types=[
            pltpu.VMEM((LANE,), jnp.int32),
            pltpu.VMEM((LANE,), jnp.int32),
            pltpu.VMEM((LANE,), jnp.int32),
            pltpu.VMEM((LANE,), jnp.int32),
            pltpu.VMEM((LANE, GRP * emb_dim), jnp.float32),
            pltpu.VMEM((LANE, GRP * emb_dim), jnp.float32),
            pltpu.VMEM((emb_dim, LANE), jnp.float32),
            pltpu.VMEM((emb_dim, LANE), jnp.float32),
            pltpu.SemaphoreType.DMA,
            pltpu.SemaphoreType.DMA,
            pltpu.SemaphoreType.DMA,
            pltpu.SemaphoreType.DMA,
        ],
        compiler_params=pltpu.CompilerParams(
            use_tc_tiling_on_sc=False, needs_layout_passes=False
        ),
    )
    def body(hi_hbm, off_hbm, table_hbm, out_hbm,
             hi_a, hi_b, off_a, off_b, grp_a, grp_b, pl_a, pl_b,
             sga, sgb, soa, sob):
        w = lax.axis_index("s") * NUM_CORES + lax.axis_index("c")
        slots = ((hi_a, off_a, grp_a, pl_a, sga, soa),
                 (hi_b, off_b, grp_b, pl_b, sgb, sob))

        def stage(h, slot):
            hi_v, off_v, grp_v, _, sg, _ = slot
            pltpu.sync_copy(hi_hbm.at[h // 8, w, h % 8], hi_v)
            pltpu.sync_copy(off_hbm.at[h // 8, w, h % 8], off_v)
            pltpu.async_copy(table_hbm.at[hi_v], grp_v, sg)

        def gather_wait(slot):
            hi_v, _, grp_v, _, sg, _ = slot
            pltpu.make_async_copy(table_hbm.at[hi_v], grp_v, sg).wait()

        def put(h, slot):
            _, _, _, pl_v, _, so = slot
            for g in range(n_dblk):
                pltpu.async_copy(
                    pl_v.at[pl.ds(8 * g, 8), :], out_hbm.at[h, g, w], so
                )

        def put_wait(h, slot):
            _, _, _, pl_v, _, so = slot
            for g in range(n_dblk):
                pltpu.make_async_copy(
                    pl_v.at[pl.ds(8 * g, 8), :], out_hbm.at[h, g, w], so
                ).wait()

        def shuffle(slot):
            _, off_v, grp_v, pl_v, _, _ = slot
            base = lax.iota(jnp.int32, 16)
            cols = [off_v[pl.ds(j0 * 16, 16)] for j0 in range(LANE // 16)]
            rows = [base + (j0 * 16) for j0 in range(LANE // 16)]

            @plsc.parallel_loop(0, emb_dim, unroll=4)
            def _(d):
                for j0 in range(LANE // 16):
                    v = plsc.load_gather(grp_v, [rows[j0], cols[j0] + d])
                    pl_v[d, pl.ds(j0 * 16, 16)] = v

        stage(0, slots[0])

        def outer(o, carry):
            for b in (0, 1):
                h = o * 2 + b
                nxt = h + 1

                @pl.when(nxt < hist)
                def _():
                    stage(nxt, slots[1 - b])

                gather_wait(slots[b])

                @pl.when(h >= 2)
                def _():
                    put_wait(h - 2, slots[b])

                shuffle(slots[b])
                put(h, slots[b])
            return carry

        lax.fori_loop(0, hist // 2, outer, 0)
        put_wait(hist - 2, slots[0])
        put_wait(hist - 1, slots[1])

    return body


def kernel(indices, table):
    batch, hist = indices.shape
    vocab, emb_dim = table.shape
    n_bblk = batch // LANE
    hist_pad = -(-hist // 8) * 8
    n_hblk = hist_pad // 8
    n_cols = vocab // LANE
    idx_p = jnp.pad(indices.astype(jnp.int32), ((0, 0), (0, hist_pad - hist)))

    def to_tiles(a):
        return a.T.reshape(n_hblk, 8, n_bblk, LANE).transpose(0, 2, 1, 3)

    hi4 = to_tiles(idx_p >> 2)
    off4 = to_tiles((idx_p & 3) << 5)
    table_t = table.T
    tail = table[n_cols * LANE:].reshape((vocab - n_cols * LANE) // GRP, LANE)
    t2 = _detile(vocab, emb_dim)(table_t, tail)
    out5 = _emb_lookup(hist, emb_dim, n_bblk)(hi4, off4, t2)
    out = out5.transpose(2, 4, 0, 1, 3).reshape(batch, hist, emb_dim)
    return out
